# Initial kernel scaffold; baseline (speedup 1.0000x reference)
import functools

import jax
import jax.numpy as jnp
from jax import lax
from jax.experimental import pallas as pl
from jax.experimental.pallas import tpu as pltpu
from jax.experimental.pallas import tpu_sc as plsc

N = 10000
D = 300
E = 160000
NCORE = 2
NSUB = 16
NPAD = 10240
CPAD = 320
HALF = CPAD // 2
CHUNK = 128
ESUB = 10240
EPAD = NSUB * ESUB
NCHUNK = ESUB // CHUNK
ROWS_PER_SUB = NPAD // NSUB
WCHUNKS = ROWS_PER_SUB // CHUNK


def _seg_sum_body(h0, h1, src, dst, out0, out1, sidx, didx, rows, shared, sem):
    c = lax.axis_index("c")
    s = lax.axis_index("s")

    def zrow(r, carry):
        for j in range(HALF // 16):
            rows[r, pl.ds(j * 16, 16)] = jnp.zeros((16,), jnp.float32)
        return carry

    lax.fori_loop(0, CHUNK, zrow, 0)
    for k in range(WCHUNKS):
        pltpu.sync_copy(rows, shared.at[pl.ds(s * ROWS_PER_SUB + k * CHUNK, CHUNK)])
    plsc.subcore_barrier()

    def body(k, carry):
        base = s * ESUB + k * CHUNK
        pltpu.sync_copy(src.at[pl.ds(base, CHUNK)], sidx)
        pltpu.sync_copy(dst.at[pl.ds(base, CHUNK)], didx)

        @pl.when(c == 0)
        def _():
            pltpu.async_copy(h0.at[sidx], rows, sem).wait()

        @pl.when(c == 1)
        def _():
            pltpu.async_copy(h1.at[sidx], rows, sem).wait()

        pltpu.sync_copy(rows, shared.at[didx], add=True)
        return carry

    lax.fori_loop(0, NCHUNK, body, 0)
    plsc.subcore_barrier()

    for k in range(WCHUNKS):
        off = s * ROWS_PER_SUB + k * CHUNK
        pltpu.sync_copy(shared.at[pl.ds(off, CHUNK)], rows)

        @pl.when(c == 0)
        def _():
            pltpu.sync_copy(rows, out0.at[pl.ds(off, CHUNK)])

        @pl.when(c == 1)
        def _():
            pltpu.sync_copy(rows, out1.at[pl.ds(off, CHUNK)])


_seg_sum = functools.partial(
    pl.kernel,
    out_type=(
        jax.ShapeDtypeStruct((NPAD, HALF), jnp.float32),
        jax.ShapeDtypeStruct((NPAD, HALF), jnp.float32),
    ),
    mesh=plsc.VectorSubcoreMesh(
        core_axis_name="c", subcore_axis_name="s",
        num_cores=NCORE, num_subcores=NSUB,
    ),
    scratch_types=[
        pltpu.VMEM((CHUNK,), jnp.int32),
        pltpu.VMEM((CHUNK,), jnp.int32),
        pltpu.VMEM((CHUNK, HALF), jnp.float32),
        pltpu.VMEM_SHARED((NPAD, HALF), jnp.float32),
        pltpu.SemaphoreType.DMA,
    ],
)(_seg_sum_body)


def _relu(x):
    return jnp.maximum(x, 0.0)


def _sigmoid(x):
    return 1.0 / (1.0 + jnp.exp(-x))


def _linear_body(act, a0, a1, wt, wb, b, o0, o1):
    x = jnp.dot(a0[...], wt[...], preferred_element_type=jnp.float32)
    x = x + jnp.dot(a1[...], wb[...], preferred_element_type=jnp.float32)
    x = x + b[...]
    x = act(x)
    o0[...] = x[:, :HALF]
    o1[...] = x[:, HALF:]


def _linear(a0, a1, wt, wb, b, act):
    rows = 1000
    return pl.pallas_call(
        functools.partial(_linear_body, act),
        grid=(N // rows,),
        in_specs=[
            pl.BlockSpec((rows, HALF), lambda i: (i, 0)),
            pl.BlockSpec((rows, HALF), lambda i: (i, 0)),
            pl.BlockSpec((HALF, CPAD), lambda i: (0, 0)),
            pl.BlockSpec((HALF, CPAD), lambda i: (0, 0)),
            pl.BlockSpec((1, CPAD), lambda i: (0, 0)),
        ],
        out_specs=[
            pl.BlockSpec((rows, HALF), lambda i: (i, 0)),
            pl.BlockSpec((rows, HALF), lambda i: (i, 0)),
        ],
        out_shape=[jax.ShapeDtypeStruct((NPAD, HALF), jnp.float32)] * 2,
    )(a0, a1, wt, wb, b)


def kernel(features, edge_index, W1, b1, W2, b2, W3, b3):
    src = edge_index[0].astype(jnp.int32)
    dst = edge_index[1].astype(jnp.int32)
    src_p = jnp.concatenate([src, jnp.zeros((EPAD - E,), jnp.int32)])
    dst_p = jnp.concatenate([dst, jnp.full((EPAD - E,), NPAD - 1, jnp.int32)])

    h = jnp.zeros((NPAD, CPAD), jnp.float32).at[:N, :D].set(features)
    h0 = h[:, :HALF]
    h1 = h[:, HALF:]

    for W, b, act in ((W1, b1, _relu), (W2, b2, _relu), (W3, b3, _sigmoid)):
        Wp = jnp.zeros((CPAD, CPAD), jnp.float32).at[:D, :D].set(W)
        bp = jnp.zeros((1, CPAD), jnp.float32).at[0, :D].set(b)
        agg0, agg1 = _seg_sum(h0, h1, src_p, dst_p)
        h0, h1 = _linear(agg0, agg1, Wp[:HALF], Wp[HALF:], bp, act)

    return jnp.concatenate([h0[:N], h1[:N, : D - HALF]], axis=1)


# speedup vs baseline: 2.3586x; 2.3586x over previous
import functools

import jax
import jax.numpy as jnp
from jax import lax
from jax.experimental import pallas as pl
from jax.experimental.pallas import tpu as pltpu
from jax.experimental.pallas import tpu_sc as plsc

N = 10000
D = 300
E = 160000
NCORE = 2
NSUB = 16
NPAD = 10240
CPAD = 320
HALF = CPAD // 2
CHUNK = 128
ESUB = 10240
EPAD = NSUB * ESUB
NCHUNK = ESUB // CHUNK
ROWS_PER_SUB = NPAD // NSUB
WCHUNKS = ROWS_PER_SUB // CHUNK


def _seg_sum_body(h0, h1, src, dst, out0, out1, sidx, didx, rows, shared, sem):
    c = lax.axis_index("c")
    s = lax.axis_index("s")

    def zrow(r, carry):
        for j in range(HALF // 16):
            rows[r, pl.ds(j * 16, 16)] = jnp.zeros((16,), jnp.float32)
        return carry

    lax.fori_loop(0, CHUNK, zrow, 0)
    for k in range(WCHUNKS):
        pltpu.sync_copy(rows, shared.at[pl.ds(s * ROWS_PER_SUB + k * CHUNK, CHUNK)])
    plsc.subcore_barrier()

    def body(k, carry):
        base = s * ESUB + k * CHUNK
        pltpu.sync_copy(src.at[pl.ds(base, CHUNK)], sidx)
        pltpu.sync_copy(dst.at[pl.ds(base, CHUNK)], didx)

        @pl.when(c == 0)
        def _():
            pltpu.async_copy(h0.at[sidx], rows, sem).wait()

        @pl.when(c == 1)
        def _():
            pltpu.async_copy(h1.at[sidx], rows, sem).wait()

        pltpu.sync_copy(rows, shared.at[didx], add=True)
        return carry

    lax.fori_loop(0, NCHUNK, body, 0)
    plsc.subcore_barrier()

    for k in range(WCHUNKS):
        off = s * ROWS_PER_SUB + k * CHUNK
        pltpu.sync_copy(shared.at[pl.ds(off, CHUNK)], rows)

        @pl.when(c == 0)
        def _():
            pltpu.sync_copy(rows, out0.at[pl.ds(off, CHUNK)])

        @pl.when(c == 1)
        def _():
            pltpu.sync_copy(rows, out1.at[pl.ds(off, CHUNK)])


@functools.cache
def _make_seg_sum():
    return functools.partial(
        pl.kernel,
        out_type=(
            jax.ShapeDtypeStruct((NPAD, HALF), jnp.float32),
            jax.ShapeDtypeStruct((NPAD, HALF), jnp.float32),
        ),
        mesh=plsc.VectorSubcoreMesh(
            core_axis_name="c", subcore_axis_name="s",
            num_cores=NCORE, num_subcores=NSUB,
        ),
        scratch_types=[
            pltpu.VMEM((CHUNK,), jnp.int32),
            pltpu.VMEM((CHUNK,), jnp.int32),
            pltpu.VMEM((CHUNK, HALF), jnp.float32),
            pltpu.VMEM_SHARED((NPAD, HALF), jnp.float32),
            pltpu.SemaphoreType.DMA,
        ],
        compiler_params=pltpu.CompilerParams(use_tc_tiling_on_sc=False),
    )(_seg_sum_body)


def _relu(x):
    return jnp.maximum(x, 0.0)


def _sigmoid(x):
    return 1.0 / (1.0 + jnp.exp(-x))


def _linear_body(act, a0, a1, wt, wb, b, o0, o1):
    x = jnp.dot(a0[...], wt[...], preferred_element_type=jnp.float32)
    x = x + jnp.dot(a1[...], wb[...], preferred_element_type=jnp.float32)
    x = x + b[...]
    x = act(x)
    o0[...] = x[:, :HALF]
    o1[...] = x[:, HALF:]


def _linear(a0, a1, wt, wb, b, act):
    rows = 1000
    return pl.pallas_call(
        functools.partial(_linear_body, act),
        grid=(N // rows,),
        in_specs=[
            pl.BlockSpec((rows, HALF), lambda i: (i, 0)),
            pl.BlockSpec((rows, HALF), lambda i: (i, 0)),
            pl.BlockSpec((HALF, CPAD), lambda i: (0, 0)),
            pl.BlockSpec((HALF, CPAD), lambda i: (0, 0)),
            pl.BlockSpec((1, CPAD), lambda i: (0, 0)),
        ],
        out_specs=[
            pl.BlockSpec((rows, HALF), lambda i: (i, 0)),
            pl.BlockSpec((rows, HALF), lambda i: (i, 0)),
        ],
        out_shape=[jax.ShapeDtypeStruct((NPAD, HALF), jnp.float32)] * 2,
    )(a0, a1, wt, wb, b)


def kernel(features, edge_index, W1, b1, W2, b2, W3, b3):
    src = edge_index[0].astype(jnp.int32)
    dst = edge_index[1].astype(jnp.int32)
    src_p = jnp.concatenate([src, jnp.zeros((EPAD - E,), jnp.int32)])
    dst_p = jnp.concatenate([dst, jnp.full((EPAD - E,), NPAD - 1, jnp.int32)])

    h = jnp.zeros((NPAD, CPAD), jnp.float32).at[:N, :D].set(features)
    h0 = h[:, :HALF]
    h1 = h[:, HALF:]

    for W, b, act in ((W1, b1, _relu), (W2, b2, _relu), (W3, b3, _sigmoid)):
        Wp = jnp.zeros((CPAD, CPAD), jnp.float32).at[:D, :D].set(W)
        bp = jnp.zeros((1, CPAD), jnp.float32).at[0, :D].set(b)
        agg0, agg1 = _make_seg_sum()(h0, h1, src_p, dst_p)
        h0, h1 = _linear(agg0, agg1, Wp[:HALF], Wp[HALF:], bp, act)

    return jnp.concatenate([h0[:N], h1[:N, : D - HALF]], axis=1)


# speedup vs baseline: 2.6449x; 1.1213x over previous
import functools

import jax
import jax.numpy as jnp
from jax import lax
from jax.experimental import pallas as pl
from jax.experimental.pallas import tpu as pltpu
from jax.experimental.pallas import tpu_sc as plsc

N = 10000
D = 300
E = 160000
NCORE = 2
NSUB = 16
NPAD = 10240
CPAD = 320
QCOL = CPAD // 4
NPASS = 2
CHUNK = 128
ESUB = 10240
EPAD = NSUB * ESUB
NCHUNK = ESUB // CHUNK
ROWS_PER_SUB = NPAD // NSUB
WCHUNKS = ROWS_PER_SUB // CHUNK
NBUF = 4
NGROUP = NCHUNK // NBUF


def _seg_sum_body(hq0, hq1, hq2, hq3, src3, dst3, oq0, oq1, oq2, oq3,
                  sall, dall, rows0, rows1, rows2, rows3, shared,
                  gsem0, gsem1, gsem2, gsem3, ssem0, ssem1, ssem2, ssem3):
    c = lax.axis_index("c")
    s = lax.axis_index("s")
    rows = (rows0, rows1, rows2, rows3)
    gsem = (gsem0, gsem1, gsem2, gsem3)
    ssem = (ssem0, ssem1, ssem2, ssem3)

    pltpu.sync_copy(src3.at[s], sall)
    pltpu.sync_copy(dst3.at[s], dall)

    def issue_gather(q, k, buf, sem):
        h_by_core = (hq0, hq2) if q == 0 else (hq1, hq3)

        @pl.when(c == 0)
        def _():
            pltpu.async_copy(h_by_core[0].at[sall.at[k]], buf, sem)

        @pl.when(c == 1)
        def _():
            pltpu.async_copy(h_by_core[1].at[sall.at[k]], buf, sem)

    def wait_gather(buf, sem):
        pltpu.make_async_copy(hq0.at[pl.ds(0, CHUNK)], buf, sem).wait()

    def wait_scatter(buf, sem):
        pltpu.make_async_copy(buf, shared.at[pl.ds(0, CHUNK)], sem).wait()

    for q in range(NPASS):
        def zrow(r, carry):
            for j in range(QCOL // 16):
                rows0[r, pl.ds(j * 16, 16)] = jnp.zeros((16,), jnp.float32)
            return carry

        lax.fori_loop(0, CHUNK, zrow, 0)
        for k in range(WCHUNKS):
            pltpu.sync_copy(
                rows0, shared.at[pl.ds(s * ROWS_PER_SUB + k * CHUNK, CHUNK)])
        plsc.subcore_barrier()

        for b in range(NBUF):
            issue_gather(q, b, rows[b], gsem[b])

        def group(i, carry):
            for b in range(NBUF):
                k = i * NBUF + b
                wait_gather(rows[b], gsem[b])
                pltpu.async_copy(rows[b], shared.at[dall.at[k]], ssem[b],
                                 add=True)
            for b in range(NBUF):
                wait_scatter(rows[b], ssem[b])

                @pl.when(i < NGROUP - 1)
                def _():
                    issue_gather(q, (i + 1) * NBUF + b, rows[b], gsem[b])
            return carry

        lax.fori_loop(0, NGROUP, group, 0)
        plsc.subcore_barrier()

        o_by_core = (oq0, oq2) if q == 0 else (oq1, oq3)
        for k in range(WCHUNKS):
            off = s * ROWS_PER_SUB + k * CHUNK
            pltpu.sync_copy(shared.at[pl.ds(off, CHUNK)], rows0)

            @pl.when(c == 0)
            def _():
                pltpu.sync_copy(rows0, o_by_core[0].at[pl.ds(off, CHUNK)])

            @pl.when(c == 1)
            def _():
                pltpu.sync_copy(rows0, o_by_core[1].at[pl.ds(off, CHUNK)])


@functools.cache
def _make_seg_sum():
    return functools.partial(
        pl.kernel,
        out_type=tuple(
            jax.ShapeDtypeStruct((NPAD, QCOL), jnp.float32) for _ in range(4)),
        mesh=plsc.VectorSubcoreMesh(
            core_axis_name="c", subcore_axis_name="s",
            num_cores=NCORE, num_subcores=NSUB,
        ),
        scratch_types=[
            pltpu.VMEM((NCHUNK, CHUNK), jnp.int32),
            pltpu.VMEM((NCHUNK, CHUNK), jnp.int32),
            pltpu.VMEM((CHUNK, QCOL), jnp.float32),
            pltpu.VMEM((CHUNK, QCOL), jnp.float32),
            pltpu.VMEM((CHUNK, QCOL), jnp.float32),
            pltpu.VMEM((CHUNK, QCOL), jnp.float32),
            pltpu.VMEM_SHARED((NPAD, QCOL), jnp.float32),
            pltpu.SemaphoreType.DMA,
            pltpu.SemaphoreType.DMA,
            pltpu.SemaphoreType.DMA,
            pltpu.SemaphoreType.DMA,
            pltpu.SemaphoreType.DMA,
            pltpu.SemaphoreType.DMA,
            pltpu.SemaphoreType.DMA,
            pltpu.SemaphoreType.DMA,
        ],
        compiler_params=pltpu.CompilerParams(use_tc_tiling_on_sc=False),
    )(_seg_sum_body)


def _relu(x):
    return jnp.maximum(x, 0.0)


def _sigmoid(x):
    return 1.0 / (1.0 + jnp.exp(-x))


def _linear_body(act, a0, a1, a2, a3, w0, w1, w2, w3, b, o0, o1, o2, o3):
    x = jnp.dot(a0[...], w0[...], preferred_element_type=jnp.float32)
    x = x + jnp.dot(a1[...], w1[...], preferred_element_type=jnp.float32)
    x = x + jnp.dot(a2[...], w2[...], preferred_element_type=jnp.float32)
    x = x + jnp.dot(a3[...], w3[...], preferred_element_type=jnp.float32)
    x = x + b[...]
    x = act(x)
    o0[...] = x[:, 0 * QCOL:1 * QCOL]
    o1[...] = x[:, 1 * QCOL:2 * QCOL]
    o2[...] = x[:, 2 * QCOL:3 * QCOL]
    o3[...] = x[:, 3 * QCOL:4 * QCOL]


def _linear(aq, wq, b, act):
    rows = 1000
    return pl.pallas_call(
        functools.partial(_linear_body, act),
        grid=(N // rows,),
        in_specs=[pl.BlockSpec((rows, QCOL), lambda i: (i, 0))] * 4
        + [pl.BlockSpec((QCOL, CPAD), lambda i: (0, 0))] * 4
        + [pl.BlockSpec((1, CPAD), lambda i: (0, 0))],
        out_specs=[pl.BlockSpec((rows, QCOL), lambda i: (i, 0))] * 4,
        out_shape=[jax.ShapeDtypeStruct((NPAD, QCOL), jnp.float32)] * 4,
    )(*aq, *wq, b)


def kernel(features, edge_index, W1, b1, W2, b2, W3, b3):
    src = edge_index[0].astype(jnp.int32)
    dst = edge_index[1].astype(jnp.int32)
    src_p = jnp.concatenate([src, jnp.zeros((EPAD - E,), jnp.int32)])
    dst_p = jnp.concatenate([dst, jnp.full((EPAD - E,), NPAD - 1, jnp.int32)])
    src3 = src_p.reshape(NSUB, NCHUNK, CHUNK)
    dst3 = dst_p.reshape(NSUB, NCHUNK, CHUNK)

    h = jnp.zeros((NPAD, CPAD), jnp.float32).at[:N, :D].set(features)
    hq = [h[:, i * QCOL:(i + 1) * QCOL] for i in range(4)]

    for W, b, act in ((W1, b1, _relu), (W2, b2, _relu), (W3, b3, _sigmoid)):
        Wp = jnp.zeros((CPAD, CPAD), jnp.float32).at[:D, :D].set(W)
        bp = jnp.zeros((1, CPAD), jnp.float32).at[0, :D].set(b)
        wq = [Wp[i * QCOL:(i + 1) * QCOL] for i in range(4)]
        aq = _make_seg_sum()(hq[0], hq[1], hq[2], hq[3], src3, dst3)
        hq = _linear(aq, wq, bp, act)

    return jnp.concatenate(
        [hq[0][:N], hq[1][:N], hq[2][:N], hq[3][:N, : D - 3 * QCOL]], axis=1)


# speedup vs baseline: 3.0087x; 1.1376x over previous
import functools

import jax
import jax.numpy as jnp
from jax import lax
from jax.experimental import pallas as pl
from jax.experimental.pallas import tpu as pltpu
from jax.experimental.pallas import tpu_sc as plsc

N = 10000
D = 300
E = 160000
NCORE = 2
NSUB = 16
NPAD = 10240
CPAD = 320
QCOL = CPAD // 4
NPASS = 2
CHUNK = 128
ESUB = 10240
EPAD = NSUB * ESUB
NCHUNK = ESUB // CHUNK
ROWS_PER_SUB = NPAD // NSUB
WCHUNKS = ROWS_PER_SUB // CHUNK
NBUF = 4
NGROUP = NCHUNK // NBUF
L1_OFFS = (0, QCOL, 2 * QCOL, D - QCOL)


def _seg_sum_body(hq0, hq1, hq2, hq3, src3, dst3, oq0, oq1, oq2, oq3,
                  sall, dall, rows0, rows1, rows2, rows3, zbuf, shared,
                  gsem0, gsem1, gsem2, gsem3, ssem0, ssem1, ssem2, ssem3,
                  zsem):
    c = lax.axis_index("c")
    s = lax.axis_index("s")
    rows = (rows0, rows1, rows2, rows3)
    gsem = (gsem0, gsem1, gsem2, gsem3)
    ssem = (ssem0, ssem1, ssem2, ssem3)

    pltpu.sync_copy(src3.at[s], sall)
    pltpu.sync_copy(dst3.at[s], dall)

    def zrow(r, carry):
        for j in range(QCOL // 16):
            zbuf[r, pl.ds(j * 16, 16)] = jnp.zeros((16,), jnp.float32)
        return carry

    lax.fori_loop(0, CHUNK, zrow, 0)

    def issue_gather(q, k, buf, sem):
        h_by_core = (hq0, hq2) if q == 0 else (hq1, hq3)

        @pl.when(c == 0)
        def _():
            pltpu.async_copy(h_by_core[0].at[sall.at[k]], buf, sem)

        @pl.when(c == 1)
        def _():
            pltpu.async_copy(h_by_core[1].at[sall.at[k]], buf, sem)

    _dummy = hq0.at[pl.ds(0, CHUNK)]

    def wait_gather(buf, sem):
        pltpu.make_async_copy(_dummy, buf, sem).wait()

    def wait_scatter(buf, sem):
        pltpu.make_async_copy(buf, shared.at[pl.ds(0, CHUNK)], sem).wait()

    for q in range(NPASS):
        for b in range(NBUF):
            issue_gather(q, b, rows[b], gsem[b])

        for k in range(WCHUNKS):
            pltpu.async_copy(
                zbuf, shared.at[pl.ds(s * ROWS_PER_SUB + k * CHUNK, CHUNK)],
                zsem)
        for k in range(WCHUNKS):
            pltpu.make_async_copy(
                zbuf, shared.at[pl.ds(0, CHUNK)], zsem).wait()
        plsc.subcore_barrier()

        def group(i, carry):
            for b in range(NBUF):
                k = i * NBUF + b
                wait_gather(rows[b], gsem[b])
                pltpu.async_copy(rows[b], shared.at[dall.at[k]], ssem[b],
                                 add=True)
            for b in range(NBUF):
                wait_scatter(rows[b], ssem[b])

                @pl.when(i < NGROUP - 1)
                def _():
                    issue_gather(q, (i + 1) * NBUF + b, rows[b], gsem[b])
            return carry

        lax.fori_loop(0, NGROUP, group, 0)
        plsc.subcore_barrier()

        o_by_core = (oq0, oq2) if q == 0 else (oq1, oq3)

        def wb_fetch(k, b):
            off = s * ROWS_PER_SUB + k * CHUNK
            pltpu.async_copy(shared.at[pl.ds(off, CHUNK)], rows[b], gsem[b])

        def wb_store(k, b):
            off = s * ROWS_PER_SUB + k * CHUNK

            @pl.when(c == 0)
            def _():
                pltpu.async_copy(rows[b], o_by_core[0].at[pl.ds(off, CHUNK)],
                                 ssem[b])

            @pl.when(c == 1)
            def _():
                pltpu.async_copy(rows[b], o_by_core[1].at[pl.ds(off, CHUNK)],
                                 ssem[b])

        def wb_wait_fetch(b):
            pltpu.make_async_copy(
                shared.at[pl.ds(0, CHUNK)], rows[b], gsem[b]).wait()

        def wb_wait_store(b):
            pltpu.make_async_copy(
                rows[b], o_by_core[0].at[pl.ds(0, CHUNK)], ssem[b]).wait()

        wb_fetch(0, 0)
        for k in range(WCHUNKS):
            b = k % 2
            wb_wait_fetch(b)
            wb_store(k, b)
            if k + 1 < WCHUNKS:
                wb_fetch(k + 1, 1 - b)
            wb_wait_store(b)


@functools.cache
def _make_seg_sum():
    return functools.partial(
        pl.kernel,
        out_type=tuple(
            jax.ShapeDtypeStruct((NPAD, QCOL), jnp.float32) for _ in range(4)),
        mesh=plsc.VectorSubcoreMesh(
            core_axis_name="c", subcore_axis_name="s",
            num_cores=NCORE, num_subcores=NSUB,
        ),
        scratch_types=[
            pltpu.VMEM((NCHUNK, CHUNK), jnp.int32),
            pltpu.VMEM((NCHUNK, CHUNK), jnp.int32),
            pltpu.VMEM((CHUNK, QCOL), jnp.float32),
            pltpu.VMEM((CHUNK, QCOL), jnp.float32),
            pltpu.VMEM((CHUNK, QCOL), jnp.float32),
            pltpu.VMEM((CHUNK, QCOL), jnp.float32),
            pltpu.VMEM((CHUNK, QCOL), jnp.float32),
            pltpu.VMEM_SHARED((NPAD, QCOL), jnp.float32),
            pltpu.SemaphoreType.DMA,
            pltpu.SemaphoreType.DMA,
            pltpu.SemaphoreType.DMA,
            pltpu.SemaphoreType.DMA,
            pltpu.SemaphoreType.DMA,
            pltpu.SemaphoreType.DMA,
            pltpu.SemaphoreType.DMA,
            pltpu.SemaphoreType.DMA,
            pltpu.SemaphoreType.DMA,
        ],
        compiler_params=pltpu.CompilerParams(use_tc_tiling_on_sc=False),
    )(_seg_sum_body)


def _relu(x):
    return jnp.maximum(x, 0.0)


def _sigmoid(x):
    return 1.0 / (1.0 + jnp.exp(-x))


def _linear_body(act, a0, a1, a2, a3, w0, w1, w2, w3, b, o0, o1, o2, o3):
    x = jnp.dot(a0[...], w0[...], preferred_element_type=jnp.float32)
    x = x + jnp.dot(a1[...], w1[...], preferred_element_type=jnp.float32)
    x = x + jnp.dot(a2[...], w2[...], preferred_element_type=jnp.float32)
    x = x + jnp.dot(a3[...], w3[...], preferred_element_type=jnp.float32)
    x = x + b[...]
    x = act(x)
    o0[...] = x[:, 0 * QCOL:1 * QCOL]
    o1[...] = x[:, 1 * QCOL:2 * QCOL]
    o2[...] = x[:, 2 * QCOL:3 * QCOL]
    o3[...] = x[:, 3 * QCOL:4 * QCOL]


def _linear_final_body(act, a0, a1, a2, a3, w0, w1, w2, w3, b, o):
    x = jnp.dot(a0[...], w0[...], preferred_element_type=jnp.float32)
    x = x + jnp.dot(a1[...], w1[...], preferred_element_type=jnp.float32)
    x = x + jnp.dot(a2[...], w2[...], preferred_element_type=jnp.float32)
    x = x + jnp.dot(a3[...], w3[...], preferred_element_type=jnp.float32)
    x = x + b[...]
    o[...] = act(x[:, :D])


def _linear(aq, wq, b, act):
    rows = 1000
    return pl.pallas_call(
        functools.partial(_linear_body, act),
        grid=(N // rows,),
        in_specs=[pl.BlockSpec((rows, QCOL), lambda i: (i, 0))] * 4
        + [pl.BlockSpec((QCOL, CPAD), lambda i: (0, 0))] * 4
        + [pl.BlockSpec((1, CPAD), lambda i: (0, 0))],
        out_specs=[pl.BlockSpec((rows, QCOL), lambda i: (i, 0))] * 4,
        out_shape=[jax.ShapeDtypeStruct((NPAD, QCOL), jnp.float32)] * 4,
    )(*aq, *wq, b)


def _linear_final(aq, wq, b, act):
    rows = 1000
    return pl.pallas_call(
        functools.partial(_linear_final_body, act),
        grid=(N // rows,),
        in_specs=[pl.BlockSpec((rows, QCOL), lambda i: (i, 0))] * 4
        + [pl.BlockSpec((QCOL, CPAD), lambda i: (0, 0))] * 4
        + [pl.BlockSpec((1, CPAD), lambda i: (0, 0))],
        out_specs=pl.BlockSpec((rows, D), lambda i: (i, 0)),
        out_shape=jax.ShapeDtypeStruct((N, D), jnp.float32),
    )(*aq, *wq, b)


def kernel(features, edge_index, W1, b1, W2, b2, W3, b3):
    src = edge_index[0].astype(jnp.int32)
    dst = edge_index[1].astype(jnp.int32)
    src_p = jnp.concatenate([src, jnp.zeros((EPAD - E,), jnp.int32)])
    dst_p = jnp.concatenate([dst, jnp.full((EPAD - E,), NPAD - 1, jnp.int32)])
    src3 = src_p.reshape(NSUB, NCHUNK, CHUNK)
    dst3 = dst_p.reshape(NSUB, NCHUNK, CHUNK)

    def wpad(W, b):
        Wp = jnp.zeros((CPAD, CPAD), jnp.float32).at[:D, :D].set(W)
        bp = jnp.zeros((1, CPAD), jnp.float32).at[0, :D].set(b)
        return Wp, bp

    fq = [lax.slice(features, (0, o), (N, o + QCOL)) for o in L1_OFFS]
    W1p, b1p = wpad(W1, b1)
    wq1 = [W1p[o:o + QCOL] for o in L1_OFFS]
    wq1[3] = wq1[3].at[: 3 * QCOL - L1_OFFS[3], :].set(0.0)
    aq = _make_seg_sum()(fq[0], fq[1], fq[2], fq[3], src3, dst3)
    hq = _linear(aq, wq1, b1p, _relu)

    W2p, b2p = wpad(W2, b2)
    wq2 = [W2p[i * QCOL:(i + 1) * QCOL] for i in range(4)]
    aq = _make_seg_sum()(hq[0], hq[1], hq[2], hq[3], src3, dst3)
    hq = _linear(aq, wq2, b2p, _relu)

    W3p, b3p = wpad(W3, b3)
    wq3 = [W3p[i * QCOL:(i + 1) * QCOL] for i in range(4)]
    aq = _make_seg_sum()(hq[0], hq[1], hq[2], hq[3], src3, dst3)
    return _linear_final(aq, wq3, b3p, _sigmoid)
